# baseline (speedup 1.0000x reference)
import functools

import jax
import jax.numpy as jnp
from jax import lax
from jax.experimental import pallas as pl
from jax.experimental.pallas import tpu as pltpu
from jax.experimental.pallas import tpu_sc as plsc

_BINS = 100
_IGNORE = 0
_R = 8
_L = 16
_NS = 16
_NC = 2


def _rowstats_block(x_ref, m_ref, s_ref):
    x = x_ref[...]
    m = jnp.max(x, axis=1)
    s = jnp.sum(jnp.exp(x - m[:, None]), axis=1)
    m_ref[0, 0, :] = m
    s_ref[0, 0, :] = s


def _row_stats(x):
    n, v = x.shape
    return pl.pallas_call(
        _rowstats_block,
        grid=(n // _R,),
        in_specs=[pl.BlockSpec((_R, v), lambda i: (i, 0))],
        out_specs=[pl.BlockSpec((1, 1, _R), lambda i: (i, 0, 0)),
                   pl.BlockSpec((1, 1, _R), lambda i: (i, 0, 0))],
        out_shape=[jax.ShapeDtypeStruct((n // _R, 1, _R), jnp.float32),
                   jax.ShapeDtypeStruct((n // _R, 1, _R), jnp.float32)],
        compiler_params=pltpu.CompilerParams(dimension_semantics=("arbitrary",)),
    )(x)


def _lane_shuffle(x, idx):
    dn = lax.GatherDimensionNumbers(
        offset_dims=(), collapsed_slice_dims=(0,), start_index_map=(0,))
    return lax.gather(x, idx[:, None], dn, slice_sizes=(1,),
                      mode=lax.GatherScatterMode.PROMISE_IN_BOUNDS)


def _lane_allsum(x, lane):
    for k in (1, 2, 4, 8):
        x = x + _lane_shuffle(x, lane ^ k)
    return x


def _sc_statistic(x_flat, ys, rmax, rsum, vocab):
    n = ys.shape[0]
    mesh = plsc.VectorSubcoreMesh(core_axis_name="c", subcore_axis_name="s")

    @functools.partial(
        pl.kernel,
        mesh=mesh,
        out_type=jax.ShapeDtypeStruct((_NC, 128), jnp.float32),
        scratch_types=[
            pltpu.VMEM((_L,), jnp.int32),
            pltpu.VMEM((_L,), jnp.float32),
            pltpu.VMEM((_L,), jnp.float32),
            pltpu.VMEM((_L,), jnp.int32),
            pltpu.VMEM((_L,), jnp.float32),
            pltpu.VMEM((128,), jnp.float32),
            pltpu.VMEM((_NS, 128), jnp.float32),
            pltpu.VMEM_SHARED((_NS, 128), jnp.float32),
            pltpu.SemaphoreType.DMA,
        ],
    )
    def sc_kernel(x_hbm, ys_hbm, max_hbm, sum_hbm, out_hbm,
                  ys_v, max_v, sum_v, idx_v, tgt_v, loc_v, red_v, sh, sem):
        c = lax.axis_index("c")
        s = lax.axis_index("s")
        base = (c * _NS + s) * _L

        pltpu.sync_copy(ys_hbm.at[pl.ds(base, _L)], ys_v)
        pltpu.sync_copy(max_hbm.at[pl.ds(base, _L)], max_v)
        pltpu.sync_copy(sum_hbm.at[pl.ds(base, _L)], sum_v)

        lane = lax.iota(jnp.int32, _L)
        ysv = ys_v[...]
        idx_v[...] = (base + lane) * vocab + ysv
        pltpu.async_copy(x_hbm.at[idx_v], tgt_v, sem).wait()

        m = max_v[...]
        se = sum_v[...]
        pv = jnp.exp(tgt_v[...] - m) / se
        valid = jnp.where(ysv != _IGNORE, 1.0, 0.0).astype(jnp.float32)

        hist = [jnp.zeros((_L,), jnp.float32) for _ in range(7)]
        bins_i = [jnp.int32(16 * j) + lane for j in range(7)]
        bins_f = [b.astype(jnp.float32) for b in bins_i]
        for r in range(_L):
            ridx = jnp.full((_L,), r, jnp.int32)
            pv_r = _lane_shuffle(pv, ridx)
            va_r = _lane_shuffle(valid, ridx)
            for j in range(7):
                upper = pv_r > bins_f[j] / _BINS
                lower = pv_r < bins_f[j] + (1.0 / _BINS)
                ok = upper & lower & (bins_i[j] < _BINS)
                hist[j] = hist[j] + jnp.where(ok, va_r, 0.0)

        sp = _lane_allsum(pv * valid, lane)
        nv = _lane_allsum(valid, lane)
        stats = jnp.where(lane == 0, sp, jnp.where(lane == 1, nv, 0.0))
        for j in range(7):
            loc_v[pl.ds(16 * j, _L)] = hist[j]
        loc_v[pl.ds(112, _L)] = stats

        pltpu.sync_copy(loc_v, sh.at[s])
        plsc.subcore_barrier()

        @pl.when(s == 0)
        def _():
            pltpu.sync_copy(sh, red_v)
            for j in range(8):
                t = red_v[0, pl.ds(16 * j, _L)]
                for w in range(1, _NS):
                    t = t + red_v[w, pl.ds(16 * j, _L)]
                loc_v[pl.ds(16 * j, _L)] = t
            pltpu.sync_copy(loc_v, out_hbm.at[c])

    return sc_kernel(x_flat, ys, rmax, rsum)


def kernel(decoder_out_att, ys_out_pad_att):
    b, t, v = decoder_out_att.shape
    n = b * t
    x = decoder_out_att.reshape(n, v)
    m, se = _row_stats(x)
    part = _sc_statistic(decoder_out_att.reshape(n * v),
                         ys_out_pad_att.reshape(n),
                         m.reshape(n), se.reshape(n), v)
    tot = part[0] + part[1]
    mean = tot[112] / jnp.maximum(tot[113], 1.0)
    return jnp.concatenate([mean[None], tot[:_BINS]])


# speedup vs baseline: 3.1417x; 3.1417x over previous
import jax
import jax.numpy as jnp
from jax.experimental import pallas as pl
from jax.experimental.pallas import tpu as pltpu

_BINS = 100
_IGNORE = 0
_R = 8


def _stat_block(ys_sref, x_ref, ys_ref, acc_ref):
    i = pl.program_id(0)

    @pl.when(i == 0)
    def _():
        acc_ref[...] = jnp.zeros_like(acc_ref)

    x = x_ref[...]
    ys = ys_ref[0, 0, :]
    m = jnp.max(x, axis=1)
    s = jnp.sum(jnp.exp(x - m[:, None]), axis=1)

    v = x.shape[1]
    v_al = (v // 128) * 128
    segs = []
    for r in range(_R):
        y_r = jnp.minimum(ys_sref[i * _R + r], v_al - 1)
        start = pl.multiple_of((y_r // 128) * 128, 128)
        segs.append(x_ref[pl.ds(r, 1), pl.ds(start, 128)])
    seg = jnp.concatenate(segs, axis=0)
    lane = jax.lax.broadcasted_iota(jnp.int32, (_R, 128), 1)
    main_lane = (jnp.minimum(ys, v_al - 1) % 128)[:, None]
    tgt = jnp.sum(jnp.where(lane == main_lane, seg, 0.0), axis=1)
    if v_al < v:
        tail = x[:, v_al:]
        tl = jax.lax.broadcasted_iota(jnp.int32, tail.shape, 1)
        tgt_tail = jnp.sum(jnp.where(tl == (ys - v_al)[:, None], tail, 0.0),
                           axis=1)
        tgt = jnp.where(ys >= v_al, tgt_tail, tgt)

    pv = jnp.exp(tgt - m) / s
    valid = (ys != _IGNORE).astype(jnp.float32)

    lanes_i = jax.lax.broadcasted_iota(jnp.int32, (_R, 128), 1)
    lanes_f = lanes_i.astype(jnp.float32)
    upper = pv[:, None] > lanes_f / _BINS
    lower = pv[:, None] < lanes_f + (1.0 / _BINS)
    mask = (upper & lower & (lanes_i < _BINS)).astype(jnp.float32) * valid[:, None]
    hist = jnp.sum(mask, axis=0)

    lane1 = jax.lax.iota(jnp.int32, 128)
    extra = jnp.where(lane1 == _BINS, jnp.sum(pv * valid),
                      jnp.where(lane1 == _BINS + 1, jnp.sum(valid), 0.0))
    acc_ref[0, :] += hist + extra


def kernel(decoder_out_att, ys_out_pad_att):
    b, t, v = decoder_out_att.shape
    n = b * t
    x = decoder_out_att.reshape(n, v)
    ys_flat = ys_out_pad_att.reshape(n)
    ys3 = ys_out_pad_att.reshape(n // _R, 1, _R)
    grid_spec = pltpu.PrefetchScalarGridSpec(
        num_scalar_prefetch=1,
        grid=(n // _R,),
        in_specs=[pl.BlockSpec((_R, v), lambda i, *_: (i, 0)),
                  pl.BlockSpec((1, 1, _R), lambda i, *_: (i, 0, 0))],
        out_specs=pl.BlockSpec((1, 128), lambda i, *_: (0, 0)),
    )
    acc = pl.pallas_call(
        _stat_block,
        grid_spec=grid_spec,
        out_shape=jax.ShapeDtypeStruct((1, 128), jnp.float32),
        compiler_params=pltpu.CompilerParams(dimension_semantics=("arbitrary",)),
    )(ys_flat, x, ys3)[0]
    mean = acc[_BINS] / jnp.maximum(acc[_BINS + 1], 1.0)
    return jnp.concatenate([mean[None], acc[:_BINS]])


# speedup vs baseline: 4.8985x; 1.5592x over previous
import jax
import jax.numpy as jnp
from jax.experimental import pallas as pl
from jax.experimental.pallas import tpu as pltpu

_BINS = 100
_IGNORE = 0
_R = 16


def _stat_block(ys_sref, x_ref, ys_ref, acc_ref):
    i = pl.program_id(0)

    @pl.when(i == 0)
    def _():
        acc_ref[...] = jnp.zeros_like(acc_ref)

    x = x_ref[...]
    ys = ys_ref[0, 0, :]
    m = jnp.max(x, axis=1)
    nk = 8
    vv = x.shape[1]
    step = ((vv // nk) // 128) * 128
    cuts = [0] + [step * (k + 1) for k in range(nk - 1)] + [vv]
    s = sum(jnp.sum(jnp.exp(x[:, c0:c1] - m[:, None]), axis=1)
            for c0, c1 in zip(cuts[:-1], cuts[1:]))

    v = x.shape[1]
    v_al = (v // 128) * 128
    segs = []
    for r in range(_R):
        y_r = jnp.minimum(ys_sref[i * _R + r], v_al - 1)
        start = pl.multiple_of((y_r // 128) * 128, 128)
        segs.append(x_ref[pl.ds(r, 1), pl.ds(start, 128)])
    seg = jnp.concatenate(segs, axis=0)
    lane = jax.lax.broadcasted_iota(jnp.int32, (_R, 128), 1)
    main_lane = (jnp.minimum(ys, v_al - 1) % 128)[:, None]
    tgt = jnp.sum(jnp.where(lane == main_lane, seg, 0.0), axis=1)
    if v_al < v:
        tail = x[:, v_al:]
        tl = jax.lax.broadcasted_iota(jnp.int32, tail.shape, 1)
        tgt_tail = jnp.sum(jnp.where(tl == (ys - v_al)[:, None], tail, 0.0),
                           axis=1)
        tgt = jnp.where(ys >= v_al, tgt_tail, tgt)

    pv = jnp.exp(tgt - m) / s
    valid = (ys != _IGNORE).astype(jnp.float32)

    lanes_i = jax.lax.broadcasted_iota(jnp.int32, (_R, 128), 1)
    lanes_f = lanes_i.astype(jnp.float32)
    upper = pv[:, None] > lanes_f / _BINS
    lower = pv[:, None] < lanes_f + (1.0 / _BINS)
    mask = (upper & lower & (lanes_i < _BINS)).astype(jnp.float32) * valid[:, None]
    hist = jnp.sum(mask, axis=0)

    lane1 = jax.lax.iota(jnp.int32, 128)
    extra = jnp.where(lane1 == _BINS, jnp.sum(pv * valid),
                      jnp.where(lane1 == _BINS + 1, jnp.sum(valid), 0.0))
    acc_ref[0, :] += hist + extra


def kernel(decoder_out_att, ys_out_pad_att):
    b, t, v = decoder_out_att.shape
    n = b * t
    x = decoder_out_att.reshape(n, v)
    ys_flat = ys_out_pad_att.reshape(n)
    ys3 = ys_out_pad_att.reshape(n // _R, 1, _R)
    grid_spec = pltpu.PrefetchScalarGridSpec(
        num_scalar_prefetch=1,
        grid=(n // _R,),
        in_specs=[pl.BlockSpec((_R, v), lambda i, *_: (i, 0)),
                  pl.BlockSpec((1, 1, _R), lambda i, *_: (i, 0, 0))],
        out_specs=pl.BlockSpec((1, 128), lambda i, *_: (0, 0)),
    )
    acc = pl.pallas_call(
        _stat_block,
        grid_spec=grid_spec,
        out_shape=jax.ShapeDtypeStruct((1, 128), jnp.float32),
        compiler_params=pltpu.CompilerParams(dimension_semantics=("arbitrary",)),
    )(ys_flat, x, ys3)[0]
    mean = acc[_BINS] / jnp.maximum(acc[_BINS + 1], 1.0)
    return jnp.concatenate([mean[None], acc[:_BINS]])


# speedup vs baseline: 5.2700x; 1.0758x over previous
import jax
import jax.numpy as jnp
from jax.experimental import pallas as pl
from jax.experimental.pallas import tpu as pltpu

_BINS = 100
_IGNORE = 0
_R = 16


def _stat_block(ys_sref, x_ref, ys_ref, acc_ref):
    i = pl.program_id(0)

    @pl.when(i == 0)
    def _():
        acc_ref[...] = jnp.zeros_like(acc_ref)

    x = x_ref[...]
    ys = ys_ref[0, 0, :]
    nk = 8
    vv = x.shape[1]
    step = ((vv // nk) // 128) * 128
    cuts = [0] + [step * (k + 1) for k in range(nk - 1)] + [vv]
    spans = list(zip(cuts[:-1], cuts[1:]))
    mparts = [jnp.max(x[:, c0:c1], axis=1) for c0, c1 in spans]
    m = mparts[0]
    for mp in mparts[1:]:
        m = jnp.maximum(m, mp)
    s = sum(jnp.sum(jnp.exp(x[:, c0:c1] - m[:, None]), axis=1)
            for c0, c1 in spans)

    v = x.shape[1]
    v_al = (v // 128) * 128
    segs = []
    for r in range(_R):
        y_r = jnp.minimum(ys_sref[i * _R + r], v_al - 1)
        start = pl.multiple_of((y_r // 128) * 128, 128)
        segs.append(x_ref[pl.ds(r, 1), pl.ds(start, 128)])
    seg = jnp.concatenate(segs, axis=0)
    lane = jax.lax.broadcasted_iota(jnp.int32, (_R, 128), 1)
    main_lane = (jnp.minimum(ys, v_al - 1) % 128)[:, None]
    tgt = jnp.sum(jnp.where(lane == main_lane, seg, 0.0), axis=1)
    if v_al < v:
        tail = x[:, v_al:]
        tl = jax.lax.broadcasted_iota(jnp.int32, tail.shape, 1)
        tgt_tail = jnp.sum(jnp.where(tl == (ys - v_al)[:, None], tail, 0.0),
                           axis=1)
        tgt = jnp.where(ys >= v_al, tgt_tail, tgt)

    pv = jnp.exp(tgt - m) / s
    valid = (ys != _IGNORE).astype(jnp.float32)

    lanes_i = jax.lax.broadcasted_iota(jnp.int32, (_R, 128), 1)
    lanes_f = lanes_i.astype(jnp.float32)
    upper = pv[:, None] > lanes_f / _BINS
    lower = pv[:, None] < lanes_f + (1.0 / _BINS)
    mask = (upper & lower & (lanes_i < _BINS)).astype(jnp.float32) * valid[:, None]
    hist = jnp.sum(mask, axis=0)

    lane1 = jax.lax.iota(jnp.int32, 128)
    extra = jnp.where(lane1 == _BINS, jnp.sum(pv * valid),
                      jnp.where(lane1 == _BINS + 1, jnp.sum(valid), 0.0))
    acc_ref[0, :] += hist + extra


def kernel(decoder_out_att, ys_out_pad_att):
    b, t, v = decoder_out_att.shape
    n = b * t
    x = decoder_out_att.reshape(n, v)
    ys_flat = ys_out_pad_att.reshape(n)
    ys3 = ys_out_pad_att.reshape(n // _R, 1, _R)
    grid_spec = pltpu.PrefetchScalarGridSpec(
        num_scalar_prefetch=1,
        grid=(n // _R,),
        in_specs=[pl.BlockSpec((_R, v), lambda i, *_: (i, 0)),
                  pl.BlockSpec((1, 1, _R), lambda i, *_: (i, 0, 0))],
        out_specs=pl.BlockSpec((1, 128), lambda i, *_: (0, 0)),
    )
    acc = pl.pallas_call(
        _stat_block,
        grid_spec=grid_spec,
        out_shape=jax.ShapeDtypeStruct((1, 128), jnp.float32),
        compiler_params=pltpu.CompilerParams(dimension_semantics=("arbitrary",)),
    )(ys_flat, x, ys3)[0]
    mean = acc[_BINS] / jnp.maximum(acc[_BINS + 1], 1.0)
    return jnp.concatenate([mean[None], acc[:_BINS]])


# speedup vs baseline: 6.1487x; 1.1667x over previous
import jax
import jax.numpy as jnp
from jax.experimental import pallas as pl
from jax.experimental.pallas import tpu as pltpu

_BINS = 100
_IGNORE = 0
_R = 32


def _stat_block(ys_sref, x_ref, ys_ref, acc_ref):
    i = pl.program_id(0)

    @pl.when(i == 0)
    def _():
        acc_ref[...] = jnp.zeros_like(acc_ref)

    x = x_ref[...]
    ys = ys_ref[0, 0, :]
    nk = 8
    vv = x.shape[1]
    step = ((vv // nk) // 128) * 128
    cuts = [0] + [step * (k + 1) for k in range(nk - 1)] + [vv]
    spans = list(zip(cuts[:-1], cuts[1:]))
    mparts = [jnp.max(x[:, c0:c1], axis=1) for c0, c1 in spans]
    m = mparts[0]
    for mp in mparts[1:]:
        m = jnp.maximum(m, mp)
    s = sum(jnp.sum(jnp.exp(x[:, c0:c1] - m[:, None]), axis=1)
            for c0, c1 in spans)

    v = x.shape[1]
    v_al = (v // 128) * 128
    segs = []
    for r in range(_R):
        y_r = jnp.minimum(ys_sref[i * _R + r], v_al - 1)
        start = pl.multiple_of((y_r // 128) * 128, 128)
        segs.append(x_ref[pl.ds(r, 1), pl.ds(start, 128)])
    seg = jnp.concatenate(segs, axis=0)
    lane = jax.lax.broadcasted_iota(jnp.int32, (_R, 128), 1)
    main_lane = (jnp.minimum(ys, v_al - 1) % 128)[:, None]
    tgt = jnp.sum(jnp.where(lane == main_lane, seg, 0.0), axis=1)
    if v_al < v:
        tail = x[:, v_al:]
        tl = jax.lax.broadcasted_iota(jnp.int32, tail.shape, 1)
        tgt_tail = jnp.sum(jnp.where(tl == (ys - v_al)[:, None], tail, 0.0),
                           axis=1)
        tgt = jnp.where(ys >= v_al, tgt_tail, tgt)

    pv = jnp.exp(tgt - m) / s
    valid = (ys != _IGNORE).astype(jnp.float32)

    lanes_i = jax.lax.broadcasted_iota(jnp.int32, (_R, 128), 1)
    lanes_f = lanes_i.astype(jnp.float32)
    upper = pv[:, None] > lanes_f / _BINS
    lower = pv[:, None] < lanes_f + (1.0 / _BINS)
    mask = (upper & lower & (lanes_i < _BINS)).astype(jnp.float32) * valid[:, None]
    hist = jnp.sum(mask, axis=0)

    lane1 = jax.lax.iota(jnp.int32, 128)
    extra = jnp.where(lane1 == _BINS, jnp.sum(pv * valid),
                      jnp.where(lane1 == _BINS + 1, jnp.sum(valid), 0.0))
    acc_ref[0, :] += hist + extra


def kernel(decoder_out_att, ys_out_pad_att):
    b, t, v = decoder_out_att.shape
    n = b * t
    x = decoder_out_att.reshape(n, v)
    ys_flat = ys_out_pad_att.reshape(n)
    ys3 = ys_out_pad_att.reshape(n // _R, 1, _R)
    grid_spec = pltpu.PrefetchScalarGridSpec(
        num_scalar_prefetch=1,
        grid=(n // _R,),
        in_specs=[pl.BlockSpec((_R, v), lambda i, *_: (i, 0)),
                  pl.BlockSpec((1, 1, _R), lambda i, *_: (i, 0, 0))],
        out_specs=pl.BlockSpec((1, 128), lambda i, *_: (0, 0)),
    )
    acc = pl.pallas_call(
        _stat_block,
        grid_spec=grid_spec,
        out_shape=jax.ShapeDtypeStruct((1, 128), jnp.float32),
        compiler_params=pltpu.CompilerParams(dimension_semantics=("arbitrary",)),
    )(ys_flat, x, ys3)[0]
    mean = acc[_BINS] / jnp.maximum(acc[_BINS + 1], 1.0)
    return jnp.concatenate([mean[None], acc[:_BINS]])
